# baseline (speedup 1.0000x reference)
import jax
import jax.numpy as jnp
from jax.experimental import pallas as pl
from jax.experimental.pallas import tpu as pltpu

_BM = 2048


def _body(vn_ref, h_ref, out_ref):
    out_ref[...] = h_ref[...] + vn_ref[...].reshape(1, vn_ref.shape[-1])


def kernel(edit_id, hidden_states, v_new, v_old, alpha, beta):
    B, S, H = hidden_states.shape
    n = B * S
    h2 = hidden_states.reshape(n, H)
    vn3 = v_new.reshape(-1, 1, H)
    out = pl.pallas_call(
        _body,
        grid=(n // _BM,),
        in_specs=[
            pl.BlockSpec((1, 1, H), lambda i: (0, 0, 0)),
            pl.BlockSpec((_BM, H), lambda i: (i, 0)),
        ],
        out_specs=pl.BlockSpec((_BM, H), lambda i: (i, 0)),
        out_shape=jax.ShapeDtypeStruct((n, H), hidden_states.dtype),
    )(vn3, h2)
    return out.reshape(B, S, H)


# speedup vs baseline: 7.5077x; 7.5077x over previous
import jax
import jax.numpy as jnp
from jax.experimental import pallas as pl
from jax.experimental.pallas import tpu as pltpu

_BM = 2048


def _body(eid_ref, vn_hbm, vo_hbm, a_hbm, b_hbm, h_ref, out_ref,
          vn_row, vo_row, a_s, b_s, ev, sem):
    i = pl.program_id(0)

    @pl.when(i == 0)
    def _gather():
        eid = eid_ref[0]
        base = (eid // 128) * 128
        c0 = pltpu.make_async_copy(vn_hbm.at[pl.ds(eid, 1), :], vn_row, sem.at[0])
        c1 = pltpu.make_async_copy(vo_hbm.at[pl.ds(eid, 1), :], vo_row, sem.at[1])
        c2 = pltpu.make_async_copy(a_hbm.at[pl.ds(base, 128)], a_s, sem.at[2])
        c3 = pltpu.make_async_copy(b_hbm.at[pl.ds(base, 128)], b_s, sem.at[3])
        c0.start(); c1.start(); c2.start(); c3.start()
        c0.wait(); c1.wait(); c2.wait(); c3.wait()
        sub = eid % 128
        ev[...] = a_s[sub] * vn_row[...] + b_s[sub] * vo_row[...]

    out_ref[...] = h_ref[...] + ev[...]


def kernel(edit_id, hidden_states, v_new, v_old, alpha, beta):
    B, S, H = hidden_states.shape
    n = B * S
    h2 = hidden_states.reshape(n, H)
    eid = jnp.asarray(edit_id, jnp.int32).reshape(1)
    pad = (-alpha.shape[0]) % 128
    a_p = jnp.pad(alpha, (0, pad))
    b_p = jnp.pad(beta, (0, pad))
    out = pl.pallas_call(
        _body,
        grid_spec=pltpu.PrefetchScalarGridSpec(
            num_scalar_prefetch=1,
            grid=(n // _BM,),
            in_specs=[
                pl.BlockSpec(memory_space=pl.ANY),
                pl.BlockSpec(memory_space=pl.ANY),
                pl.BlockSpec(memory_space=pl.ANY),
                pl.BlockSpec(memory_space=pl.ANY),
                pl.BlockSpec((_BM, H), lambda i, e: (i, 0)),
            ],
            out_specs=pl.BlockSpec((_BM, H), lambda i, e: (i, 0)),
            scratch_shapes=[
                pltpu.VMEM((1, H), jnp.float32),
                pltpu.VMEM((1, H), jnp.float32),
                pltpu.SMEM((128,), jnp.float32),
                pltpu.SMEM((128,), jnp.float32),
                pltpu.VMEM((1, H), jnp.float32),
                pltpu.SemaphoreType.DMA((4,)),
            ],
        ),
        out_shape=jax.ShapeDtypeStruct((n, H), hidden_states.dtype),
    )(eid, v_new, v_old, a_p, b_p, h2)
    return out.reshape(B, S, H)
